# Initial kernel scaffold; baseline (speedup 1.0000x reference)
import jax
import jax.numpy as jnp
from jax.experimental import pallas as pl


def kernel(x, edge_index, W1, b1, W2, b2):
    raise NotImplementedError("write your pallas kernel here")


# speedup vs baseline: 13.1074x; 13.1074x over previous
import functools

import jax
import jax.numpy as jnp
from jax import lax
from jax.experimental import pallas as pl
from jax.experimental.pallas import tpu as pltpu
from jax.experimental.pallas import tpu_sc as plsc

NC = 2
NS = 16
LANES = 16


def _pad_rows(n_nodes):
    per = -(-n_nodes // (NS * 128)) * 128
    return per * NS, per


def _deg_partials(dst, n_nodes):
    e = dst.shape[0]
    ept = e // (NC * NS)
    ch = 80
    nchunk = ept // ch
    assert ept % ch == 0 and e % (NC * NS) == 0
    n_pad, rpt = _pad_rows(n_nodes)
    zr = 128
    assert rpt % zr == 0

    mesh = plsc.VectorSubcoreMesh(core_axis_name="c", subcore_axis_name="s")

    @functools.partial(
        pl.kernel,
        out_type=jax.ShapeDtypeStruct((NC, n_pad, LANES), jnp.float32),
        mesh=mesh,
        scratch_types=[
            pltpu.VMEM_SHARED((n_pad, LANES), jnp.float32),
            pltpu.VMEM((ch,), jnp.int32),
            pltpu.VMEM((ch, LANES), jnp.float32),
            pltpu.VMEM((zr, LANES), jnp.float32),
        ],
    )
    def k(dst_hbm, out_hbm, sdeg, idxd, ones, zbuf):
        c = lax.axis_index("c")
        s = lax.axis_index("s")

        zeros16 = jnp.zeros((LANES,), jnp.float32)
        ones16 = jnp.ones((LANES,), jnp.float32)

        @pl.loop(0, zr)
        def _(r):
            zbuf[r, :] = zeros16

        @pl.loop(0, ch)
        def _(r):
            ones[r, :] = ones16

        @pl.loop(0, rpt // zr)
        def _(j):
            pltpu.sync_copy(zbuf, sdeg.at[pl.ds(s * rpt + j * zr, zr)])

        plsc.subcore_barrier()

        base0 = c * (e // NC) + s * ept

        @pl.loop(0, nchunk)
        def _(i):
            pltpu.sync_copy(dst_hbm.at[pl.ds(base0 + i * ch, ch)], idxd)
            pltpu.sync_copy(ones, sdeg.at[idxd], add=True)

        plsc.subcore_barrier()

        pltpu.sync_copy(
            sdeg.at[pl.ds(s * rpt, rpt)],
            out_hbm.at[c, pl.ds(s * rpt, rpt)],
        )

    return k(dst)


def _edge_partials(hs, src, dst):
    n_nodes, d = hs.shape
    e = src.shape[0]
    ept = e // (NC * NS)
    ch = 80
    nchunk = ept // ch
    assert ept % ch == 0
    n_pad, rpt = _pad_rows(n_nodes)
    zr = 128
    assert rpt % zr == 0

    mesh = plsc.VectorSubcoreMesh(core_axis_name="c", subcore_axis_name="s")

    @functools.partial(
        pl.kernel,
        out_type=jax.ShapeDtypeStruct((NC, n_pad, d), jnp.float32),
        mesh=mesh,
        scratch_types=[
            pltpu.VMEM_SHARED((n_pad, d), jnp.float32),
            pltpu.VMEM((ch,), jnp.int32),
            pltpu.VMEM((ch,), jnp.int32),
            pltpu.VMEM((ch, d), jnp.float32),
            pltpu.VMEM((zr, d), jnp.float32),
            pltpu.SemaphoreType.DMA,
        ],
    )
    def k(hs_hbm, src_hbm, dst_hbm, out_hbm, acc, idxs, idxd, rows, zbuf, sem):
        c = lax.axis_index("c")
        s = lax.axis_index("s")

        zeros16 = jnp.zeros((LANES,), jnp.float32)

        @pl.loop(0, zr)
        def _(r):
            @pl.loop(0, d // LANES)
            def _(j):
                zbuf[r, pl.ds(j * LANES, LANES)] = zeros16

        @pl.loop(0, rpt // zr)
        def _(j):
            pltpu.sync_copy(zbuf, acc.at[pl.ds(s * rpt + j * zr, zr)])

        plsc.subcore_barrier()

        base0 = c * (e // NC) + s * ept

        @pl.loop(0, nchunk)
        def _(i):
            pltpu.sync_copy(src_hbm.at[pl.ds(base0 + i * ch, ch)], idxs)
            pltpu.sync_copy(dst_hbm.at[pl.ds(base0 + i * ch, ch)], idxd)
            pltpu.async_copy(hs_hbm.at[idxs], rows, sem).wait()
            pltpu.sync_copy(rows, acc.at[idxd], add=True)

        plsc.subcore_barrier()

        pltpu.sync_copy(
            acc.at[pl.ds(s * rpt, rpt)],
            out_hbm.at[c, pl.ds(s * rpt, rpt)],
        )

    return k(hs, src, dst)


_RB = 1000


def _dis_block(degp):
    deg = degp[0, :, 0:1] + degp[1, :, 0:1] + 1.0
    return lax.rsqrt(deg)


def _mm_scale(x, w, degp):
    n, din = x.shape
    dout = w.shape[1]
    grid = (n // _RB,)

    def body(x_ref, w_ref, degp_ref, o_ref):
        dis = _dis_block(degp_ref[...])
        o_ref[...] = (
            jnp.dot(x_ref[...], w_ref[...], preferred_element_type=jnp.float32) * dis
        )

    return pl.pallas_call(
        body,
        grid=grid,
        in_specs=[
            pl.BlockSpec((_RB, din), lambda i: (i, 0)),
            pl.BlockSpec((din, dout), lambda i: (0, 0)),
            pl.BlockSpec((2, _RB, 16), lambda i: (0, i, 0)),
        ],
        out_specs=pl.BlockSpec((_RB, dout), lambda i: (i, 0)),
        out_shape=jax.ShapeDtypeStruct((n, dout), jnp.float32),
    )(x, w, degp)


def _combine_mm(p, hs, degp, b, w):
    n, d = hs.shape
    dout = w.shape[1]
    grid = (n // _RB,)

    def body(p_ref, hs_ref, degp_ref, b_ref, w_ref, o_ref):
        dis = _dis_block(degp_ref[...])
        y = (p_ref[0] + p_ref[1] + hs_ref[...]) * dis + b_ref[...]
        y = jnp.maximum(y, 0.0)
        o_ref[...] = (
            jnp.dot(y, w_ref[...], preferred_element_type=jnp.float32) * dis
        )

    return pl.pallas_call(
        body,
        grid=grid,
        in_specs=[
            pl.BlockSpec((2, _RB, d), lambda i: (0, i, 0)),
            pl.BlockSpec((_RB, d), lambda i: (i, 0)),
            pl.BlockSpec((2, _RB, 16), lambda i: (0, i, 0)),
            pl.BlockSpec((1, d), lambda i: (0, 0)),
            pl.BlockSpec((d, dout), lambda i: (0, 0)),
        ],
        out_specs=pl.BlockSpec((_RB, dout), lambda i: (i, 0)),
        out_shape=jax.ShapeDtypeStruct((n, dout), jnp.float32),
    )(p, hs, degp, b, w)


def _combine_final(p, hs, degp, b):
    n, d = hs.shape
    grid = (n // _RB,)

    def body(p_ref, hs_ref, degp_ref, b_ref, o_ref):
        dis = _dis_block(degp_ref[...])
        y = (p_ref[0] + p_ref[1] + hs_ref[...]) * dis + b_ref[...]
        o_ref[...] = jnp.maximum(y, 0.0)

    return pl.pallas_call(
        body,
        grid=grid,
        in_specs=[
            pl.BlockSpec((2, _RB, d), lambda i: (0, i, 0)),
            pl.BlockSpec((_RB, d), lambda i: (i, 0)),
            pl.BlockSpec((2, _RB, 16), lambda i: (0, i, 0)),
            pl.BlockSpec((1, d), lambda i: (0, 0)),
        ],
        out_specs=pl.BlockSpec((_RB, d), lambda i: (i, 0)),
        out_shape=jax.ShapeDtypeStruct((n, d), jnp.float32),
    )(p, hs, degp, b)


def kernel(x, edge_index, W1, b1, W2, b2):
    n_nodes = x.shape[0]
    ei = edge_index.astype(jnp.int32)
    src = ei[0]
    dst = ei[1]

    degp = _deg_partials(dst, n_nodes)
    hs1 = _mm_scale(x, W1, degp)
    p1 = _edge_partials(hs1, src, dst)
    hs2 = _combine_mm(p1, hs1, degp, b1.reshape(1, -1), W2)
    p2 = _edge_partials(hs2, src, dst)
    out = _combine_final(p2, hs2, degp, b2.reshape(1, -1))
    return out


# speedup vs baseline: 28.3026x; 2.1593x over previous
import functools

import jax
import jax.numpy as jnp
from jax import lax
from jax.experimental import pallas as pl
from jax.experimental.pallas import tpu as pltpu
from jax.experimental.pallas import tpu_sc as plsc

NC = 2
NS = 16
LANES = 16


def _pad_rows(n_nodes):
    per = -(-n_nodes // (NS * 128)) * 128
    return per * NS, per


_CH = 80


def _deg_partials(dst, n_nodes):
    e = dst.shape[0]
    ept = e // (NC * NS)
    ch = 80
    nchunk = ept // ch
    assert ept % ch == 0
    n_pad, rpt = _pad_rows(n_nodes)
    zr = 128
    assert rpt % zr == 0

    mesh = plsc.VectorSubcoreMesh(core_axis_name="c", subcore_axis_name="s")

    @functools.partial(
        pl.kernel,
        out_type=jax.ShapeDtypeStruct((NC, n_pad, LANES), jnp.float32),
        mesh=mesh,
        scratch_types=[
            pltpu.VMEM_SHARED((n_pad, LANES), jnp.float32),
            pltpu.VMEM((ch,), jnp.int32),
            pltpu.VMEM((ch, LANES), jnp.float32),
            pltpu.VMEM((zr, LANES), jnp.float32),
        ],
    )
    def k(dst_hbm, out_hbm, sdeg, idxd, ones, zbuf):
        c = lax.axis_index("c")
        s = lax.axis_index("s")

        zeros16 = jnp.zeros((LANES,), jnp.float32)
        ones16 = jnp.ones((LANES,), jnp.float32)

        @pl.loop(0, zr)
        def _(r):
            zbuf[r, :] = zeros16

        @pl.loop(0, ch)
        def _(r):
            ones[r, :] = ones16

        @pl.loop(0, rpt // zr)
        def _(j):
            pltpu.sync_copy(zbuf, sdeg.at[pl.ds(s * rpt + j * zr, zr)])

        plsc.subcore_barrier()

        base0 = c * (e // NC) + s * ept

        @pl.loop(0, nchunk)
        def _(i):
            pltpu.sync_copy(dst_hbm.at[pl.ds(base0 + i * ch, ch)], idxd)
            pltpu.sync_copy(ones, sdeg.at[idxd], add=True)

        plsc.subcore_barrier()

        pltpu.sync_copy(
            sdeg.at[pl.ds(s * rpt, rpt)],
            out_hbm.at[c, pl.ds(s * rpt, rpt)],
        )

    return k(dst)


_BISECT_SIMPLE = False
_NBUF = 3
_LEADG = 2
_DRING = 8
_LEADD = 4
_UNROLL = 24


def _edge_partials(hs, src, dst, zrows):
    n_nodes, d = hs.shape
    e = src.shape[0]
    ept = e // (NC * NS)
    ch = _CH
    nchunk = ept // ch
    assert ept % ch == 0
    n_pad, rpt = _pad_rows(n_nodes)
    assert nchunk > _UNROLL

    mesh = plsc.VectorSubcoreMesh(core_axis_name="c", subcore_axis_name="s")

    @functools.partial(
        pl.kernel,
        out_type=jax.ShapeDtypeStruct((NC, n_pad, d), jnp.float32),
        mesh=mesh,
        scratch_types=(
            [
                pltpu.VMEM_SHARED((n_pad, d), jnp.float32),
                pltpu.VMEM((ept,), jnp.int32),
                pltpu.VMEM((_DRING, ch), jnp.int32),
            ]
            + [pltpu.VMEM((ch, d), jnp.float32)] * _NBUF
            + [pltpu.SemaphoreType.DMA] * (2 * _NBUF + 1 + _DRING)
        ),
    )
    def k(hs_hbm, src_hbm, dst_hbm, zrows_hbm, out_hbm, acc, idxs, dring, *bufsem):
        rows = bufsem[:_NBUF]
        gsem = bufsem[_NBUF : 2 * _NBUF]
        ssem = bufsem[2 * _NBUF : 3 * _NBUF]
        isem = bufsem[3 * _NBUF]
        dsem = bufsem[3 * _NBUF + 1 :]

        c = lax.axis_index("c")
        s = lax.axis_index("s")
        tbase = (c * NS + s) * ept

        pltpu.async_copy(src_hbm.at[pl.ds(tbase, ept)], idxs, isem)
        if not _BISECT_SIMPLE:
            for j in range(_LEADD):
                pltpu.async_copy(
                    dst_hbm.at[pl.ds(tbase + j * ch, ch)], dring.at[j], dsem[j]
                )
        pltpu.sync_copy(zrows_hbm, acc.at[pl.ds(s * rpt, rpt)])
        pltpu.make_async_copy(src_hbm.at[pl.ds(tbase, ept)], idxs, isem).wait()
        plsc.subcore_barrier()

        if not _BISECT_SIMPLE:
            for b in range(_LEADG):
                pltpu.async_copy(
                    hs_hbm.at[idxs.at[pl.ds(b * ch, ch)]], rows[b], gsem[b]
                )

        def step(i, b24):
            b = b24 % _NBUF
            sl = b24 % _DRING
            pltpu.make_async_copy(
                hs_hbm.at[idxs.at[pl.ds(0, ch)]], rows[b], gsem[b]
            ).wait()
            pltpu.make_async_copy(
                dst_hbm.at[pl.ds(0, ch)], dring.at[sl], dsem[sl]
            ).wait()
            pltpu.async_copy(rows[b], acc.at[dring.at[sl]], ssem[b], add=True)

            bj = (b24 + _LEADG) % _NBUF
            j1 = i + _LEADG

            @pl.when(j1 < nchunk)
            def _():
                @pl.when(j1 >= _NBUF)
                def _():
                    pltpu.make_async_copy(
                        rows[bj], acc.at[dring.at[0]], ssem[bj]
                    ).wait()

                pltpu.async_copy(
                    hs_hbm.at[idxs.at[pl.ds(j1 * ch, ch)]], rows[bj], gsem[bj]
                )

            sl2 = (b24 + _LEADD) % _DRING
            j2 = i + _LEADD

            @pl.when(j2 < nchunk)
            def _():
                pltpu.async_copy(
                    dst_hbm.at[pl.ds(tbase + j2 * ch, ch)], dring.at[sl2],
                    dsem[sl2],
                )

        ngroup = nchunk // _UNROLL

        if _BISECT_SIMPLE:
            @pl.loop(0, nchunk)
            def _(i):
                pltpu.sync_copy(dst_hbm.at[pl.ds(tbase + i * ch, ch)], dring.at[0])
                pltpu.sync_copy(src_hbm.at[pl.ds(tbase + i * ch, ch)], dring.at[1])
                pltpu.async_copy(
                    hs_hbm.at[dring.at[1]], rows[0], gsem[0]
                ).wait()
                pltpu.sync_copy(rows[0], acc.at[dring.at[0]], add=True)
        else:
            @pl.loop(0, ngroup)
            def _(g):
                for b24 in range(_UNROLL):
                    step(g * _UNROLL + b24, b24)

            for b24 in range(nchunk % _UNROLL):
                step(ngroup * _UNROLL + b24, b24)

        if not _BISECT_SIMPLE:
            for b in range(_NBUF):
                pltpu.make_async_copy(rows[b], acc.at[dring.at[0]], ssem[b]).wait()

        plsc.subcore_barrier()

        pltpu.sync_copy(
            acc.at[pl.ds(s * rpt, rpt)],
            out_hbm.at[c, pl.ds(s * rpt, rpt)],
        )

    return k(hs, src, dst, zrows)


_RB = 1000


def _dis_block(degp):
    deg = degp[0, :, 0:1] + degp[1, :, 0:1] + 1.0
    return lax.rsqrt(deg)


def _mm_scale(x, w, degp):
    n, din = x.shape
    dout = w.shape[1]
    grid = (n // _RB,)

    def body(x_ref, w_ref, degp_ref, o_ref):
        dis = _dis_block(degp_ref[...])
        o_ref[...] = (
            jnp.dot(x_ref[...], w_ref[...], preferred_element_type=jnp.float32) * dis
        )

    return pl.pallas_call(
        body,
        grid=grid,
        in_specs=[
            pl.BlockSpec((_RB, din), lambda i: (i, 0)),
            pl.BlockSpec((din, dout), lambda i: (0, 0)),
            pl.BlockSpec((2, _RB, 16), lambda i: (0, i, 0)),
        ],
        out_specs=pl.BlockSpec((_RB, dout), lambda i: (i, 0)),
        out_shape=jax.ShapeDtypeStruct((n, dout), jnp.float32),
    )(x, w, degp)


def _combine_mm(p, hs, degp, b, w):
    n, d = hs.shape
    dout = w.shape[1]
    grid = (n // _RB,)

    def body(p_ref, hs_ref, degp_ref, b_ref, w_ref, o_ref):
        dis = _dis_block(degp_ref[...])
        y = (p_ref[0] + p_ref[1] + hs_ref[...]) * dis + b_ref[...]
        y = jnp.maximum(y, 0.0)
        o_ref[...] = (
            jnp.dot(y, w_ref[...], preferred_element_type=jnp.float32) * dis
        )

    return pl.pallas_call(
        body,
        grid=grid,
        in_specs=[
            pl.BlockSpec((2, _RB, d), lambda i: (0, i, 0)),
            pl.BlockSpec((_RB, d), lambda i: (i, 0)),
            pl.BlockSpec((2, _RB, 16), lambda i: (0, i, 0)),
            pl.BlockSpec((1, d), lambda i: (0, 0)),
            pl.BlockSpec((d, dout), lambda i: (0, 0)),
        ],
        out_specs=pl.BlockSpec((_RB, dout), lambda i: (i, 0)),
        out_shape=jax.ShapeDtypeStruct((n, dout), jnp.float32),
    )(p, hs, degp, b, w)


def _combine_final(p, hs, degp, b):
    n, d = hs.shape
    grid = (n // _RB,)

    def body(p_ref, hs_ref, degp_ref, b_ref, o_ref):
        dis = _dis_block(degp_ref[...])
        y = (p_ref[0] + p_ref[1] + hs_ref[...]) * dis + b_ref[...]
        o_ref[...] = jnp.maximum(y, 0.0)

    return pl.pallas_call(
        body,
        grid=grid,
        in_specs=[
            pl.BlockSpec((2, _RB, d), lambda i: (0, i, 0)),
            pl.BlockSpec((_RB, d), lambda i: (i, 0)),
            pl.BlockSpec((2, _RB, 16), lambda i: (0, i, 0)),
            pl.BlockSpec((1, d), lambda i: (0, 0)),
        ],
        out_specs=pl.BlockSpec((_RB, d), lambda i: (i, 0)),
        out_shape=jax.ShapeDtypeStruct((n, d), jnp.float32),
    )(p, hs, degp, b)


def kernel(x, edge_index, W1, b1, W2, b2):
    n_nodes = x.shape[0]
    e = edge_index.shape[1]
    ept = e // (NC * NS)
    assert ept % _CH == 0
    nchunk = ept // _CH
    ei = edge_index.astype(jnp.int32)
    src = ei[0]
    dst = ei[1]

    _, rpt = _pad_rows(n_nodes)
    d = W1.shape[1]
    zdeg = jnp.zeros((rpt, LANES), jnp.float32)
    zrows = jnp.zeros((rpt, d), jnp.float32)

    degp = _deg_partials(dst, n_nodes)
    hs1 = _mm_scale(x, W1, degp)
    p1 = _edge_partials(hs1, src, dst, zrows)
    hs2 = _combine_mm(p1, hs1, degp, b1.reshape(1, -1), W2)
    p2 = _edge_partials(hs2, src, dst, zrows)
    out = _combine_final(p2, hs2, degp, b2.reshape(1, -1))
    return out


# speedup vs baseline: 32.3325x; 1.1424x over previous
import functools

import jax
import jax.numpy as jnp
from jax import lax
from jax.experimental import pallas as pl
from jax.experimental.pallas import tpu as pltpu
from jax.experimental.pallas import tpu_sc as plsc

NC = 2
NS = 16
LANES = 16


def _pad_rows(n_nodes):
    per = -(-n_nodes // (NS * 128)) * 128
    return per * NS, per


_CH = 80


def _deg_partials(dst, zdeg, n_nodes):
    e = dst.shape[0]
    ept = e // (NC * NS)
    ch = _CH
    nchunk = ept // ch
    assert ept % ch == 0
    n_pad, rpt = _pad_rows(n_nodes)

    mesh = plsc.VectorSubcoreMesh(core_axis_name="c", subcore_axis_name="s")

    @functools.partial(
        pl.kernel,
        out_type=jax.ShapeDtypeStruct((NC, n_pad, LANES), jnp.float32),
        mesh=mesh,
        scratch_types=(
            [
                pltpu.VMEM_SHARED((n_pad, LANES), jnp.float32),
                pltpu.VMEM((_DRING, ch), jnp.int32),
                pltpu.VMEM((ch, LANES), jnp.float32),
            ]
            + [pltpu.SemaphoreType.DMA] * (2 * _DRING)
        ),
    )
    def k(dst_hbm, zdeg_hbm, out_hbm, sdeg, dring, ones, *sems):
        dsem = sems[:_DRING]
        ssem = sems[_DRING:]

        c = lax.axis_index("c")
        s = lax.axis_index("s")
        tbase = (c * NS + s) * ept

        for j in range(_LEADD):
            pltpu.async_copy(
                dst_hbm.at[pl.ds(tbase + j * ch, ch)], dring.at[j], dsem[j]
            )

        ones16 = jnp.ones((LANES,), jnp.float32)

        @pl.loop(0, ch)
        def _(r):
            ones[r, :] = ones16

        pltpu.sync_copy(zdeg_hbm, sdeg.at[pl.ds(s * rpt, rpt)])
        plsc.subcore_barrier()

        def step(i, b8):
            sl = b8 % _DRING
            pltpu.make_async_copy(
                dst_hbm.at[pl.ds(0, ch)], dring.at[sl], dsem[sl]
            ).wait()
            pltpu.async_copy(ones, sdeg.at[dring.at[sl]], ssem[sl], add=True)

            sl2 = (b8 + _LEADD) % _DRING
            j2 = i + _LEADD

            @pl.when(j2 < nchunk)
            def _():
                @pl.when(j2 >= _DRING)
                def _():
                    pltpu.make_async_copy(
                        ones, sdeg.at[dring.at[0]], ssem[sl2]
                    ).wait()

                pltpu.async_copy(
                    dst_hbm.at[pl.ds(tbase + j2 * ch, ch)], dring.at[sl2],
                    dsem[sl2],
                )

        ngroup = nchunk // _DRING

        @pl.loop(0, ngroup)
        def _(g):
            for b8 in range(_DRING):
                step(g * _DRING + b8, b8)

        for b8 in range(nchunk % _DRING):
            step(ngroup * _DRING + b8, b8)

        for b8 in range(_DRING):
            pltpu.make_async_copy(ones, sdeg.at[dring.at[0]], ssem[b8]).wait()

        plsc.subcore_barrier()

        pltpu.sync_copy(
            sdeg.at[pl.ds(s * rpt, rpt)],
            out_hbm.at[c, pl.ds(s * rpt, rpt)],
        )

    return k(dst, zdeg)


_BISECT_SIMPLE = False
_NBUF = 3
_LEADG = 2
_DRING = 8
_LEADD = 4
_UNROLL = 24


def _edge_partials(hs, src, dst, zrows):
    n_nodes, d = hs.shape
    e = src.shape[0]
    ept = e // (NC * NS)
    ch = _CH
    nchunk = ept // ch
    assert ept % ch == 0
    n_pad, rpt = _pad_rows(n_nodes)
    assert nchunk > _UNROLL

    mesh = plsc.VectorSubcoreMesh(core_axis_name="c", subcore_axis_name="s")

    @functools.partial(
        pl.kernel,
        out_type=jax.ShapeDtypeStruct((NC, n_pad, d), jnp.float32),
        mesh=mesh,
        scratch_types=(
            [
                pltpu.VMEM_SHARED((n_pad, d), jnp.float32),
                pltpu.VMEM((ept,), jnp.int32),
                pltpu.VMEM((_DRING, ch), jnp.int32),
            ]
            + [pltpu.VMEM((ch, d), jnp.float32)] * _NBUF
            + [pltpu.SemaphoreType.DMA] * (2 * _NBUF + 1 + _DRING)
        ),
    )
    def k(hs_hbm, src_hbm, dst_hbm, zrows_hbm, out_hbm, acc, idxs, dring, *bufsem):
        rows = bufsem[:_NBUF]
        gsem = bufsem[_NBUF : 2 * _NBUF]
        ssem = bufsem[2 * _NBUF : 3 * _NBUF]
        isem = bufsem[3 * _NBUF]
        dsem = bufsem[3 * _NBUF + 1 :]

        c = lax.axis_index("c")
        s = lax.axis_index("s")
        tbase = (c * NS + s) * ept

        pltpu.async_copy(src_hbm.at[pl.ds(tbase, ept)], idxs, isem)
        if not _BISECT_SIMPLE:
            for j in range(_LEADD):
                pltpu.async_copy(
                    dst_hbm.at[pl.ds(tbase + j * ch, ch)], dring.at[j], dsem[j]
                )
        pltpu.sync_copy(zrows_hbm, acc.at[pl.ds(s * rpt, rpt)])
        pltpu.make_async_copy(src_hbm.at[pl.ds(tbase, ept)], idxs, isem).wait()
        plsc.subcore_barrier()

        if not _BISECT_SIMPLE:
            for b in range(_LEADG):
                pltpu.async_copy(
                    hs_hbm.at[idxs.at[pl.ds(b * ch, ch)]], rows[b], gsem[b]
                )

        def step(i, b24):
            b = b24 % _NBUF
            sl = b24 % _DRING
            pltpu.make_async_copy(
                hs_hbm.at[idxs.at[pl.ds(0, ch)]], rows[b], gsem[b]
            ).wait()
            pltpu.make_async_copy(
                dst_hbm.at[pl.ds(0, ch)], dring.at[sl], dsem[sl]
            ).wait()
            pltpu.async_copy(rows[b], acc.at[dring.at[sl]], ssem[b], add=True)

            bj = (b24 + _LEADG) % _NBUF
            j1 = i + _LEADG

            @pl.when(j1 < nchunk)
            def _():
                @pl.when(j1 >= _NBUF)
                def _():
                    pltpu.make_async_copy(
                        rows[bj], acc.at[dring.at[0]], ssem[bj]
                    ).wait()

                pltpu.async_copy(
                    hs_hbm.at[idxs.at[pl.ds(j1 * ch, ch)]], rows[bj], gsem[bj]
                )

            sl2 = (b24 + _LEADD) % _DRING
            j2 = i + _LEADD

            @pl.when(j2 < nchunk)
            def _():
                pltpu.async_copy(
                    dst_hbm.at[pl.ds(tbase + j2 * ch, ch)], dring.at[sl2],
                    dsem[sl2],
                )

        ngroup = nchunk // _UNROLL

        if _BISECT_SIMPLE:
            @pl.loop(0, nchunk)
            def _(i):
                pltpu.sync_copy(dst_hbm.at[pl.ds(tbase + i * ch, ch)], dring.at[0])
                pltpu.sync_copy(src_hbm.at[pl.ds(tbase + i * ch, ch)], dring.at[1])
                pltpu.async_copy(
                    hs_hbm.at[dring.at[1]], rows[0], gsem[0]
                ).wait()
                pltpu.sync_copy(rows[0], acc.at[dring.at[0]], add=True)
        else:
            @pl.loop(0, ngroup)
            def _(g):
                for b24 in range(_UNROLL):
                    step(g * _UNROLL + b24, b24)

            for b24 in range(nchunk % _UNROLL):
                step(ngroup * _UNROLL + b24, b24)

        if not _BISECT_SIMPLE:
            for b in range(_NBUF):
                pltpu.make_async_copy(rows[b], acc.at[dring.at[0]], ssem[b]).wait()

        plsc.subcore_barrier()

        pltpu.sync_copy(
            acc.at[pl.ds(s * rpt, rpt)],
            out_hbm.at[c, pl.ds(s * rpt, rpt)],
        )

    return k(hs, src, dst, zrows)


_RB = 1000


def _dis_block(degp):
    deg = degp[0, :, 0:1] + degp[1, :, 0:1] + 1.0
    return lax.rsqrt(deg)


def _mm_scale(x, w, degp):
    n, din = x.shape
    dout = w.shape[1]
    grid = (n // _RB,)

    def body(x_ref, w_ref, degp_ref, o_ref):
        dis = _dis_block(degp_ref[...])
        o_ref[...] = (
            jnp.dot(x_ref[...], w_ref[...], preferred_element_type=jnp.float32) * dis
        )

    return pl.pallas_call(
        body,
        grid=grid,
        in_specs=[
            pl.BlockSpec((_RB, din), lambda i: (i, 0)),
            pl.BlockSpec((din, dout), lambda i: (0, 0)),
            pl.BlockSpec((2, _RB, 16), lambda i: (0, i, 0)),
        ],
        out_specs=pl.BlockSpec((_RB, dout), lambda i: (i, 0)),
        out_shape=jax.ShapeDtypeStruct((n, dout), jnp.float32),
    )(x, w, degp)


def _combine_mm(p, hs, degp, b, w):
    n, d = hs.shape
    dout = w.shape[1]
    grid = (n // _RB,)

    def body(p_ref, hs_ref, degp_ref, b_ref, w_ref, o_ref):
        dis = _dis_block(degp_ref[...])
        y = (p_ref[0] + p_ref[1] + hs_ref[...]) * dis + b_ref[...]
        y = jnp.maximum(y, 0.0)
        o_ref[...] = (
            jnp.dot(y, w_ref[...], preferred_element_type=jnp.float32) * dis
        )

    return pl.pallas_call(
        body,
        grid=grid,
        in_specs=[
            pl.BlockSpec((2, _RB, d), lambda i: (0, i, 0)),
            pl.BlockSpec((_RB, d), lambda i: (i, 0)),
            pl.BlockSpec((2, _RB, 16), lambda i: (0, i, 0)),
            pl.BlockSpec((1, d), lambda i: (0, 0)),
            pl.BlockSpec((d, dout), lambda i: (0, 0)),
        ],
        out_specs=pl.BlockSpec((_RB, dout), lambda i: (i, 0)),
        out_shape=jax.ShapeDtypeStruct((n, dout), jnp.float32),
    )(p, hs, degp, b, w)


def _combine_final(p, hs, degp, b):
    n, d = hs.shape
    grid = (n // _RB,)

    def body(p_ref, hs_ref, degp_ref, b_ref, o_ref):
        dis = _dis_block(degp_ref[...])
        y = (p_ref[0] + p_ref[1] + hs_ref[...]) * dis + b_ref[...]
        o_ref[...] = jnp.maximum(y, 0.0)

    return pl.pallas_call(
        body,
        grid=grid,
        in_specs=[
            pl.BlockSpec((2, _RB, d), lambda i: (0, i, 0)),
            pl.BlockSpec((_RB, d), lambda i: (i, 0)),
            pl.BlockSpec((2, _RB, 16), lambda i: (0, i, 0)),
            pl.BlockSpec((1, d), lambda i: (0, 0)),
        ],
        out_specs=pl.BlockSpec((_RB, d), lambda i: (i, 0)),
        out_shape=jax.ShapeDtypeStruct((n, d), jnp.float32),
    )(p, hs, degp, b)


def kernel(x, edge_index, W1, b1, W2, b2):
    n_nodes = x.shape[0]
    e = edge_index.shape[1]
    ept = e // (NC * NS)
    assert ept % _CH == 0
    nchunk = ept // _CH
    ei = edge_index.astype(jnp.int32)
    src = ei[0]
    dst = ei[1]

    _, rpt = _pad_rows(n_nodes)
    d = W1.shape[1]
    zdeg = jnp.zeros((rpt, LANES), jnp.float32)
    zrows = jnp.zeros((rpt, d), jnp.float32)

    degp = _deg_partials(dst, zdeg, n_nodes)
    hs1 = _mm_scale(x, W1, degp)
    p1 = _edge_partials(hs1, src, dst, zrows)
    hs2 = _combine_mm(p1, hs1, degp, b1.reshape(1, -1), W2)
    p2 = _edge_partials(hs2, src, dst, zrows)
    out = _combine_final(p2, hs2, degp, b2.reshape(1, -1))
    return out
